# Initial kernel scaffold; baseline (speedup 1.0000x reference)
import functools

import jax
import jax.numpy as jnp
from jax import lax
from jax.experimental import pallas as pl
from jax.experimental.pallas import tpu as pltpu
from jax.experimental.pallas import tpu_sc as plsc

B = 16384
L = 50
ID_DIM = 32
TEXT_DIM = 32
OUT_DIM = ID_DIM + TEXT_DIM

NC, NS = 2, 16
NW = NC * NS
UPW = B // NW
GW = 128
NJ = UPW // GW
KT = 10
NB = L // KT


def _make_kernel(interpret=False):
    mesh = plsc.VectorSubcoreMesh(core_axis_name="c", subcore_axis_name="s")

    @functools.partial(
        pl.kernel,
        out_type=jax.ShapeDtypeStruct((NW, NJ, GW, OUT_DIM), jnp.float32),
        mesh=mesh,
        interpret=interpret,
        scratch_types=[
            pltpu.VMEM((L, NJ, GW), jnp.int32),
            pltpu.VMEM((NJ, GW), jnp.int32),
            pltpu.VMEM((NJ, GW, ID_DIM), jnp.float32),
            pltpu.VMEM((KT, GW, TEXT_DIM), jnp.float32),
            pltpu.VMEM((GW, TEXT_DIM), jnp.float32),
            pltpu.VMEM((UPW,), jnp.float32),
            pltpu.VMEM((UPW,), jnp.float32),
            pltpu.VMEM((TEXT_DIM,), jnp.float32),
            pltpu.VMEM((NJ, GW, TEXT_DIM), jnp.float32),
            pltpu.SemaphoreType.DMA,
            pltpu.SemaphoreType.DMA,
        ],
    )
    def user_model(idx3, tokw, id_tab, txt_tab, out,
                   tok_v, idv, idrows, rows_v, acc, cnt, recip, row0, pooled,
                   sem, sem2):
        w = lax.axis_index("s") * NC + lax.axis_index("c")

        pltpu.sync_copy(tokw.at[w], tok_v)
        pltpu.sync_copy(idx3.at[w], idv)
        pltpu.sync_copy(txt_tab.at[0], row0)

        iddescs = [
            pltpu.async_copy(id_tab.at[idv.at[j]], idrows.at[j], sem2)
            for j in range(NJ)
        ]

        @pl.loop(0, NJ)
        def _cnt_loop(j):
            for g in range(GW // 16):
                def body(t, c):
                    tok = tok_v[t, j, pl.ds(g * 16, 16)]
                    return c + jnp.where(tok == 0, 1.0, 0.0)
                c = lax.fori_loop(0, L, body, jnp.zeros((16,), jnp.float32))
                off = pl.multiple_of(j * GW + g * 16, 16)
                cnt[pl.ds(off, 16)] = c
                recip[pl.ds(off, 16)] = 1.0 / jnp.maximum(
                    jnp.float32(L) - c, 1.0)

        r0a = row0[pl.ds(0, 16)]
        r0b = row0[pl.ds(16, 16)]

        @pl.loop(0, NJ)
        def _j_loop(j):
            @pl.loop(0, GW)
            def _zero(u):
                z = jnp.zeros((16,), jnp.float32)
                acc[u, pl.ds(0, 16)] = z
                acc[u, pl.ds(16, 16)] = z

            @pl.loop(0, NB)
            def _batch(b):
                descs = [
                    pltpu.async_copy(
                        txt_tab.at[tok_v.at[b * KT + t, j]], rows_v.at[t], sem)
                    for t in range(KT)
                ]
                for d in descs:
                    d.wait()

                @pl.loop(0, GW)
                def _reduce(u):
                    h0 = acc[u, pl.ds(0, 16)]
                    h1 = acc[u, pl.ds(16, 16)]
                    for t in range(KT):
                        h0 = h0 + rows_v[t, u, pl.ds(0, 16)]
                        h1 = h1 + rows_v[t, u, pl.ds(16, 16)]
                    acc[u, pl.ds(0, 16)] = h0
                    acc[u, pl.ds(16, 16)] = h1

            @pl.loop(0, GW)
            def _fin(u):
                ub = jnp.full((16,), j * GW + u, jnp.int32)
                cb = plsc.load_gather(cnt, [ub])
                rb = plsc.load_gather(recip, [ub])
                pooled[j, u, pl.ds(0, 16)] = (acc[u, pl.ds(0, 16)] - cb * r0a) * rb
                pooled[j, u, pl.ds(16, 16)] = (acc[u, pl.ds(16, 16)] - cb * r0b) * rb

        for d in iddescs:
            d.wait()
        pltpu.sync_copy(idrows, out.at[w, :, :, pl.ds(0, ID_DIM)])
        pltpu.sync_copy(pooled, out.at[w, :, :, pl.ds(ID_DIM, TEXT_DIM)])

    return user_model


_user_model = _make_kernel()


def kernel(id_indices, token_ids, id_table, text_table):
    idx3 = id_indices.reshape(NW, NJ, GW).astype(jnp.int32)
    tokw = token_ids.astype(jnp.int32).T.reshape(L, NW, NJ, GW).transpose(1, 0, 2, 3)
    out = _user_model(idx3, tokw, id_table, text_table)
    return out.reshape(B, OUT_DIM)


# speedup vs baseline: 19.4065x; 19.4065x over previous
import functools

import jax
import jax.numpy as jnp
from jax import lax
from jax.experimental import pallas as pl
from jax.experimental.pallas import tpu as pltpu
from jax.experimental.pallas import tpu_sc as plsc

B = 16384
L = 50
ID_DIM = 32
TEXT_DIM = 32
OUT_DIM = ID_DIM + TEXT_DIM

NC, NS = 2, 16
NW = NC * NS
UPW = B // NW
GW = 128
NJ = UPW // GW
KT = 10
NB = L // KT


def _make_kernel(interpret=False):
    mesh = plsc.VectorSubcoreMesh(core_axis_name="c", subcore_axis_name="s")

    @functools.partial(
        pl.kernel,
        out_type=jax.ShapeDtypeStruct((NW, NJ, GW, OUT_DIM), jnp.float32),
        mesh=mesh,
        interpret=interpret,
        compiler_params=pltpu.CompilerParams(use_tc_tiling_on_sc=False),
        scratch_types=[
            pltpu.VMEM((L, NJ, GW), jnp.int32),
            pltpu.VMEM((NJ, GW), jnp.int32),
            pltpu.VMEM((NJ, GW, OUT_DIM), jnp.float32),
            pltpu.VMEM((NJ, GW, ID_DIM), jnp.float32),
            pltpu.VMEM((KT, GW, TEXT_DIM), jnp.float32),
            pltpu.VMEM((GW, TEXT_DIM), jnp.float32),
            pltpu.VMEM((UPW,), jnp.float32),
            pltpu.VMEM((UPW,), jnp.float32),
            pltpu.VMEM((TEXT_DIM,), jnp.float32),
            pltpu.SemaphoreType.DMA,
            pltpu.SemaphoreType.DMA,
        ],
    )
    def user_model(idx3, tokw, id_tab, txt_tab, out,
                   tok_v, idv, outbuf, idrows, rows_v, acc, cnt, recip, row0,
                   sem, sem2):
        w = lax.axis_index("s") * NC + lax.axis_index("c")

        pltpu.sync_copy(tokw.at[w], tok_v)
        pltpu.sync_copy(idx3.at[w], idv)
        pltpu.sync_copy(txt_tab.at[0], row0)

        iddescs = [
            pltpu.async_copy(id_tab.at[idv.at[j]], idrows.at[j], sem2)
            for j in range(NJ)
        ]

        @pl.loop(0, NJ)
        def _cnt_loop(j):
            for g in range(GW // 16):
                def body(t, c):
                    tok = tok_v[t, j, pl.ds(g * 16, 16)]
                    return c + jnp.where(tok == 0, 1.0, 0.0)
                c = lax.fori_loop(0, L, body, jnp.zeros((16,), jnp.float32))
                off = pl.multiple_of(j * GW + g * 16, 16)
                cnt[pl.ds(off, 16)] = c
                recip[pl.ds(off, 16)] = 1.0 / jnp.maximum(
                    jnp.float32(L) - c, 1.0)

        r0a = row0[pl.ds(0, 16)]
        r0b = row0[pl.ds(16, 16)]

        for d in iddescs:
            d.wait()

        @pl.loop(0, NJ)
        def _j_loop(j):
            @pl.loop(0, GW)
            def _zero(u):
                z = jnp.zeros((16,), jnp.float32)
                acc[u, pl.ds(0, 16)] = z
                acc[u, pl.ds(16, 16)] = z

            @pl.loop(0, NB)
            def _batch(b):
                descs = [
                    pltpu.async_copy(
                        txt_tab.at[tok_v.at[b * KT + t, j]], rows_v.at[t], sem)
                    for t in range(KT)
                ]
                for d in descs:
                    d.wait()

                @pl.loop(0, GW)
                def _reduce(u):
                    h0 = acc[u, pl.ds(0, 16)]
                    h1 = acc[u, pl.ds(16, 16)]
                    for t in range(KT):
                        h0 = h0 + rows_v[t, u, pl.ds(0, 16)]
                        h1 = h1 + rows_v[t, u, pl.ds(16, 16)]
                    acc[u, pl.ds(0, 16)] = h0
                    acc[u, pl.ds(16, 16)] = h1

            @pl.loop(0, GW // 16)
            def _fin(g):
                off = pl.multiple_of(j * GW + g * 16, 16)
                cg = cnt[pl.ds(off, 16)]
                rg = recip[pl.ds(off, 16)]
                for u16 in range(16):
                    u = g * 16 + u16
                    cb = jnp.full((16,), cg[u16], jnp.float32)
                    rb = jnp.full((16,), rg[u16], jnp.float32)
                    outbuf[j, u, pl.ds(0, 16)] = idrows[j, u, pl.ds(0, 16)]
                    outbuf[j, u, pl.ds(16, 16)] = idrows[j, u, pl.ds(16, 16)]
                    outbuf[j, u, pl.ds(ID_DIM, 16)] = (
                        acc[u, pl.ds(0, 16)] - cb * r0a) * rb
                    outbuf[j, u, pl.ds(ID_DIM + 16, 16)] = (
                        acc[u, pl.ds(16, 16)] - cb * r0b) * rb

        pltpu.sync_copy(outbuf, out.at[w])

    return user_model


_user_model = _make_kernel()


def kernel(id_indices, token_ids, id_table, text_table):
    idx3 = id_indices.reshape(NW, NJ, GW).astype(jnp.int32)
    tokw = token_ids.astype(jnp.int32).T.reshape(L, NW, NJ, GW).transpose(1, 0, 2, 3)
    out = _user_model(idx3, tokw, id_table, text_table)
    return out.reshape(B, OUT_DIM)


# speedup vs baseline: 22.7646x; 1.1730x over previous
import functools

import jax
import jax.numpy as jnp
from jax import lax
from jax.experimental import pallas as pl
from jax.experimental.pallas import tpu as pltpu
from jax.experimental.pallas import tpu_sc as plsc

B = 16384
L = 50
ID_DIM = 32
TEXT_DIM = 32
OUT_DIM = ID_DIM + TEXT_DIM

NC, NS = 2, 16
NW = NC * NS
UPW = B // NW
GW = 128
NJ = UPW // GW
KT = 5
NB = L // KT


def _make_kernel(interpret=False):
    mesh = plsc.VectorSubcoreMesh(core_axis_name="c", subcore_axis_name="s")

    @functools.partial(
        pl.kernel,
        out_type=jax.ShapeDtypeStruct((NW, NJ, GW, OUT_DIM), jnp.float32),
        mesh=mesh,
        interpret=interpret,
        compiler_params=pltpu.CompilerParams(use_tc_tiling_on_sc=False),
        scratch_types=[
            pltpu.VMEM((L, NJ, GW), jnp.int32),
            pltpu.VMEM((NJ, GW), jnp.int32),
            pltpu.VMEM((NJ, GW, OUT_DIM), jnp.float32),
            pltpu.VMEM((NJ, GW, ID_DIM), jnp.float32),
            pltpu.VMEM((KT, GW, TEXT_DIM), jnp.float32),
            pltpu.VMEM((KT, GW, TEXT_DIM), jnp.float32),
            pltpu.VMEM((GW, TEXT_DIM), jnp.float32),
            pltpu.VMEM((UPW,), jnp.float32),
            pltpu.VMEM((UPW,), jnp.float32),
            pltpu.VMEM((TEXT_DIM,), jnp.float32),
            pltpu.SemaphoreType.DMA,
            pltpu.SemaphoreType.DMA,
            pltpu.SemaphoreType.DMA,
        ],
    )
    def user_model(idx3, tokw, id_tab, txt_tab, out,
                   tok_v, idv, outbuf, idrows, rows_a, rows_b, acc, cnt,
                   recip, row0, sem_a, sem_b, sem2):
        w = lax.axis_index("s") * NC + lax.axis_index("c")

        pltpu.sync_copy(tokw.at[w], tok_v)
        pltpu.sync_copy(idx3.at[w], idv)
        pltpu.sync_copy(txt_tab.at[0], row0)

        iddescs = [
            pltpu.async_copy(id_tab.at[idv.at[j]], idrows.at[j], sem2)
            for j in range(NJ)
        ]

        @pl.loop(0, NJ)
        def _cnt_loop(j):
            for g in range(GW // 16):
                def body(t, c):
                    tok = tok_v[t, j, pl.ds(g * 16, 16)]
                    return c + jnp.where(tok == 0, 1.0, 0.0)
                c = lax.fori_loop(0, L, body, jnp.zeros((16,), jnp.float32))
                off = pl.multiple_of(j * GW + g * 16, 16)
                cnt[pl.ds(off, 16)] = c
                recip[pl.ds(off, 16)] = 1.0 / jnp.maximum(
                    jnp.float32(L) - c, 1.0)

        r0a = row0[pl.ds(0, 16)]
        r0b = row0[pl.ds(16, 16)]

        for d in iddescs:
            d.wait()

        @pl.loop(0, NJ)
        def _j_loop(j):
            @pl.loop(0, GW)
            def _zero(u):
                z = jnp.zeros((16,), jnp.float32)
                acc[u, pl.ds(0, 16)] = z
                acc[u, pl.ds(16, 16)] = z

            def _fire(b, buf, sem):
                return [
                    pltpu.async_copy(
                        txt_tab.at[tok_v.at[b * KT + t, j]], buf.at[t], sem)
                    for t in range(KT)
                ]

            def _drain(buf, sem):
                for t in range(KT):
                    pltpu.make_async_copy(
                        txt_tab.at[tok_v.at[t, j]], buf.at[t], sem).wait()

            def _reduce_batch(buf):
                @pl.loop(0, GW)
                def _reduce(u):
                    h0 = acc[u, pl.ds(0, 16)]
                    h1 = acc[u, pl.ds(16, 16)]
                    for t in range(KT):
                        h0 = h0 + buf[t, u, pl.ds(0, 16)]
                        h1 = h1 + buf[t, u, pl.ds(16, 16)]
                    acc[u, pl.ds(0, 16)] = h0
                    acc[u, pl.ds(16, 16)] = h1

            _fire(0, rows_a, sem_a)

            @pl.loop(0, NB // 2)
            def _pair(p):
                _fire(2 * p + 1, rows_b, sem_b)
                _drain(rows_a, sem_a)
                _reduce_batch(rows_a)

                @pl.when(p < NB // 2 - 1)
                def _():
                    _fire(2 * p + 2, rows_a, sem_a)

                _drain(rows_b, sem_b)
                _reduce_batch(rows_b)

            @pl.loop(0, GW // 16)
            def _fin(g):
                off = pl.multiple_of(j * GW + g * 16, 16)
                cg = cnt[pl.ds(off, 16)]
                rg = recip[pl.ds(off, 16)]
                for u16 in range(16):
                    u = g * 16 + u16
                    cb = jnp.full((16,), cg[u16], jnp.float32)
                    rb = jnp.full((16,), rg[u16], jnp.float32)
                    outbuf[j, u, pl.ds(0, 16)] = idrows[j, u, pl.ds(0, 16)]
                    outbuf[j, u, pl.ds(16, 16)] = idrows[j, u, pl.ds(16, 16)]
                    outbuf[j, u, pl.ds(ID_DIM, 16)] = (
                        acc[u, pl.ds(0, 16)] - cb * r0a) * rb
                    outbuf[j, u, pl.ds(ID_DIM + 16, 16)] = (
                        acc[u, pl.ds(16, 16)] - cb * r0b) * rb

        pltpu.sync_copy(outbuf, out.at[w])

    return user_model


_user_model = _make_kernel()


def kernel(id_indices, token_ids, id_table, text_table):
    idx3 = id_indices.reshape(NW, NJ, GW).astype(jnp.int32)
    tokw = token_ids.astype(jnp.int32).T.reshape(L, NW, NJ, GW).transpose(1, 0, 2, 3)
    out = _user_model(idx3, tokw, id_table, text_table)
    return out.reshape(B, OUT_DIM)


# speedup vs baseline: 24.2736x; 1.0663x over previous
import functools

import jax
import jax.numpy as jnp
from jax import lax
from jax.experimental import pallas as pl
from jax.experimental.pallas import tpu as pltpu
from jax.experimental.pallas import tpu_sc as plsc

B = 16384
L = 50
ID_DIM = 32
TEXT_DIM = 32
OUT_DIM = ID_DIM + TEXT_DIM

NC, NS = 2, 16
NW = NC * NS
UPW = B // NW
GW = 128
NJ = UPW // GW
KT = 5
NB = L // KT


def _make_kernel(interpret=False):
    mesh = plsc.VectorSubcoreMesh(core_axis_name="c", subcore_axis_name="s")

    @functools.partial(
        pl.kernel,
        out_type=jax.ShapeDtypeStruct((NW, NJ, GW, OUT_DIM), jnp.float32),
        mesh=mesh,
        interpret=interpret,
        compiler_params=pltpu.CompilerParams(use_tc_tiling_on_sc=False),
        scratch_types=[
            pltpu.VMEM((L, NJ, GW), jnp.int32),
            pltpu.VMEM((NJ, GW), jnp.int32),
            pltpu.VMEM((NJ, GW, OUT_DIM), jnp.float32),
            pltpu.VMEM((NJ, GW, ID_DIM), jnp.float32),
            pltpu.VMEM((KT, GW, TEXT_DIM), jnp.float32),
            pltpu.VMEM((KT, GW, TEXT_DIM), jnp.float32),
            pltpu.VMEM((GW, TEXT_DIM), jnp.float32),
            pltpu.VMEM((UPW,), jnp.float32),
            pltpu.VMEM((UPW,), jnp.float32),
            pltpu.VMEM((TEXT_DIM,), jnp.float32),
            pltpu.SemaphoreType.DMA,
            pltpu.SemaphoreType.DMA,
            pltpu.SemaphoreType.DMA,
        ],
    )
    def user_model(idx3, tokw, id_tab, txt_tab, out,
                   tok_v, idv, outbuf, idrows, rows_a, rows_b, acc, cnt,
                   recip, row0, sem_a, sem_b, sem2):
        w = lax.axis_index("s") * NC + lax.axis_index("c")

        pltpu.sync_copy(tokw.at[w], tok_v)
        pltpu.sync_copy(idx3.at[w], idv)
        pltpu.sync_copy(txt_tab.at[0], row0)

        iddescs = [
            pltpu.async_copy(id_tab.at[idv.at[j]], idrows.at[j], sem2)
            for j in range(NJ)
        ]

        @pl.loop(0, NJ)
        def _cnt_loop(j):
            for g in range(GW // 16):
                def body(t, c):
                    tok = tok_v[t, j, pl.ds(g * 16, 16)]
                    return c + jnp.where(tok == 0, 1.0, 0.0)
                c = lax.fori_loop(0, L, body, jnp.zeros((16,), jnp.float32),
                                  unroll=5)
                off = pl.multiple_of(j * GW + g * 16, 16)
                cnt[pl.ds(off, 16)] = c
                recip[pl.ds(off, 16)] = 1.0 / jnp.maximum(
                    jnp.float32(L) - c, 1.0)

        r0a = row0[pl.ds(0, 16)]
        r0b = row0[pl.ds(16, 16)]

        for d in iddescs:
            d.wait()

        @pl.loop(0, NJ)
        def _j_loop(j):
            @pl.loop(0, GW, unroll=8)
            def _zero(u):
                z = jnp.zeros((16,), jnp.float32)
                acc[u, pl.ds(0, 16)] = z
                acc[u, pl.ds(16, 16)] = z

            def _fire(b, buf, sem):
                return [
                    pltpu.async_copy(
                        txt_tab.at[tok_v.at[b * KT + t, j]], buf.at[t], sem)
                    for t in range(KT)
                ]

            def _drain(buf, sem):
                for t in range(KT):
                    pltpu.make_async_copy(
                        txt_tab.at[tok_v.at[t, j]], buf.at[t], sem).wait()

            def _reduce_batch(buf):
                @pl.loop(0, GW, unroll=4)
                def _reduce(u):
                    h0 = acc[u, pl.ds(0, 16)]
                    h1 = acc[u, pl.ds(16, 16)]
                    for t in range(KT):
                        h0 = h0 + buf[t, u, pl.ds(0, 16)]
                        h1 = h1 + buf[t, u, pl.ds(16, 16)]
                    acc[u, pl.ds(0, 16)] = h0
                    acc[u, pl.ds(16, 16)] = h1

            _fire(0, rows_a, sem_a)

            @pl.loop(0, NB // 2)
            def _pair(p):
                _fire(2 * p + 1, rows_b, sem_b)
                _drain(rows_a, sem_a)
                _reduce_batch(rows_a)

                @pl.when(p < NB // 2 - 1)
                def _():
                    _fire(2 * p + 2, rows_a, sem_a)

                _drain(rows_b, sem_b)
                _reduce_batch(rows_b)

            @pl.loop(0, GW // 16)
            def _fin(g):
                off = pl.multiple_of(j * GW + g * 16, 16)
                cg = cnt[pl.ds(off, 16)]
                rg = recip[pl.ds(off, 16)]
                for u16 in range(16):
                    u = g * 16 + u16
                    cb = jnp.full((16,), cg[u16], jnp.float32)
                    rb = jnp.full((16,), rg[u16], jnp.float32)
                    outbuf[j, u, pl.ds(0, 16)] = idrows[j, u, pl.ds(0, 16)]
                    outbuf[j, u, pl.ds(16, 16)] = idrows[j, u, pl.ds(16, 16)]
                    outbuf[j, u, pl.ds(ID_DIM, 16)] = (
                        acc[u, pl.ds(0, 16)] - cb * r0a) * rb
                    outbuf[j, u, pl.ds(ID_DIM + 16, 16)] = (
                        acc[u, pl.ds(16, 16)] - cb * r0b) * rb

        pltpu.sync_copy(outbuf, out.at[w])

    return user_model


_user_model = _make_kernel()


def kernel(id_indices, token_ids, id_table, text_table):
    idx3 = id_indices.reshape(NW, NJ, GW).astype(jnp.int32)
    tokw = token_ids.astype(jnp.int32).T.reshape(L, NW, NJ, GW).transpose(1, 0, 2, 3)
    out = _user_model(idx3, tokw, id_table, text_table)
    return out.reshape(B, OUT_DIM)
